# Initial kernel scaffold; baseline (speedup 1.0000x reference)
import jax
import jax.numpy as jnp
from jax.experimental import pallas as pl


def kernel(input_ids_0, input_ids_1, word_emb, pos_emb, seg_emb, ln_weight, ln_bias):
    raise NotImplementedError("write your pallas kernel here")


# speedup vs baseline: 1.1076x; 1.1076x over previous
import functools

import jax
import jax.numpy as jnp
from jax import lax
from jax.experimental import pallas as pl
from jax.experimental.pallas import tpu as pltpu
from jax.experimental.pallas import tpu_sc as plsc

HIDDEN = 128
SEQ = 2048
SEG_LEN = 1024
PAD_WORD = 0
NC = 2
NS = 16
NW = NC * NS
BPW = SEQ // NW
NCH = HIDDEN // 16
EPS = 1e-5
MAGIC = 0x5F3759DF


def _emb_ln_body(ids_hbm, wemb_hbm, pemb_hbm, semb_hbm, lnw_hbm, lnb_hbm,
                 out_hbm, mask_hbm,
                 idx_v, rows_v, pos_v, seg_v, lnw_v, lnb_v, mask_v, sem):
    wid = lax.axis_index("s") * NC + lax.axis_index("c")
    base = wid * BPW

    pltpu.sync_copy(ids_hbm.at[pl.ds(base, BPW)], idx_v)
    gather = pltpu.async_copy(wemb_hbm.at[idx_v], rows_v, sem)

    pos_base = lax.rem(base, SEG_LEN)
    pltpu.sync_copy(pemb_hbm.at[pl.ds(pos_base, BPW)], pos_v)
    seg_id = base // SEG_LEN
    pltpu.sync_copy(semb_hbm.at[pl.ds(seg_id, 1)], seg_v)
    pltpu.sync_copy(lnw_hbm, lnw_v)
    pltpu.sync_copy(lnb_hbm, lnb_v)

    for j in range(BPW // 16):
        ids16 = idx_v[pl.ds(j * 16, 16)]
        mask_v[pl.ds(j * 16, 16)] = jnp.where(
            ids16 == PAD_WORD, jnp.int32(1), jnp.int32(0))
    pltpu.sync_copy(mask_v, mask_hbm.at[pl.ds(base, BPW)])

    segs = [seg_v[0, pl.ds(c * 16, 16)] for c in range(NCH)]
    lnws = [lnw_v[pl.ds(c * 16, 16)] for c in range(NCH)]
    lnbs = [lnb_v[pl.ds(c * 16, 16)] for c in range(NCH)]
    inv_d = jnp.float32(1.0 / HIDDEN)

    gather.wait()

    lanes = lax.iota(jnp.int32, 16)
    perms = [lax.bitwise_xor(lanes, jnp.int32(s)) for s in (8, 4, 2, 1)]

    def hsum(t):
        for idx in perms:
            t = t + t.at[idx].get(mode="promise_in_bounds",
                                  unique_indices=True)
        return t

    def row_body(i, carry):
        vs = []
        for c in range(NCH):
            w = rows_v[i, pl.ds(c * 16, 16)]
            p = pos_v[i, pl.ds(c * 16, 16)]
            vs.append(w + p + segs[c])
        t0 = (vs[0] + vs[1]) + (vs[2] + vs[3])
        t1 = (vs[4] + vs[5]) + (vs[6] + vs[7])
        mean = hsum(t0 + t1) * inv_d
        cen = [v - mean for v in vs]
        sq = [v * v for v in cen]
        s0 = (sq[0] + sq[1]) + (sq[2] + sq[3])
        s1 = (sq[4] + sq[5]) + (sq[6] + sq[7])
        x = hsum(s0 + s1) * inv_d + EPS
        iv = lax.bitcast_convert_type(x, jnp.int32)
        y = lax.bitcast_convert_type(
            jnp.int32(MAGIC) - lax.shift_right_logical(iv, 1), jnp.float32)
        half_x = x * 0.5
        for _ in range(3):
            y = y * (1.5 - half_x * y * y)
        for c in range(NCH):
            rows_v[i, pl.ds(c * 16, 16)] = cen[c] * y * lnws[c] + lnbs[c]
        return carry

    lax.fori_loop(0, BPW, row_body, 0, unroll=False)
    pltpu.sync_copy(rows_v, out_hbm.at[pl.ds(base, BPW)])


@jax.jit
def _emb_ln(ids, word_emb, pos_emb, seg_emb, ln_weight, ln_bias):
    mesh = plsc.VectorSubcoreMesh(
        core_axis_name="c", subcore_axis_name="s", num_cores=NC)
    return pl.kernel(
        _emb_ln_body,
        out_type=[
            jax.ShapeDtypeStruct((SEQ, HIDDEN), jnp.float32),
            jax.ShapeDtypeStruct((SEQ,), jnp.int32),
        ],
        mesh=mesh,
        scratch_types=[
            pltpu.VMEM((BPW,), jnp.int32),
            pltpu.VMEM((BPW, HIDDEN), jnp.float32),
            pltpu.VMEM((BPW, HIDDEN), jnp.float32),
            pltpu.VMEM((1, HIDDEN), jnp.float32),
            pltpu.VMEM((HIDDEN,), jnp.float32),
            pltpu.VMEM((HIDDEN,), jnp.float32),
            pltpu.VMEM((BPW,), jnp.int32),
            pltpu.SemaphoreType.DMA,
        ],
    )(ids, word_emb, pos_emb, seg_emb, ln_weight, ln_bias)


def kernel(input_ids_0, input_ids_1, word_emb, pos_emb, seg_emb, ln_weight,
           ln_bias):
    ids = jnp.concatenate([input_ids_0[0], input_ids_1[0, 1:]])
    out, mask = _emb_ln(ids, word_emb, pos_emb, seg_emb, ln_weight, ln_bias)
    return out[None], (mask != 0)[None]


# speedup vs baseline: 1.1174x; 1.0088x over previous
import functools

import jax
import jax.numpy as jnp
from jax import lax
from jax.experimental import pallas as pl
from jax.experimental.pallas import tpu as pltpu
from jax.experimental.pallas import tpu_sc as plsc

HIDDEN = 128
SEQ = 2048
SEG_LEN = 1024
PAD_WORD = 0
NC = 2
NS = 16
NW = NC * NS
BPW = SEQ // NW
NCH = HIDDEN // 16
EPS = 1e-5
MAGIC = 0x5F3759DF


def _emb_ln_body(ids_hbm, wemb_hbm, pemb_hbm, semb_hbm, lnw_hbm, lnb_hbm,
                 out_hbm, mask_hbm,
                 idx_v, rows_v, pos_v, seg_v, lnw_v, lnb_v, mask_v, sem):
    wid = lax.axis_index("s") * NC + lax.axis_index("c")
    base = wid * BPW

    pltpu.sync_copy(ids_hbm.at[pl.ds(base, BPW)], idx_v)
    gather = pltpu.async_copy(wemb_hbm.at[idx_v], rows_v, sem)

    pos_base = lax.rem(base, SEG_LEN)
    pltpu.sync_copy(pemb_hbm.at[pl.ds(pos_base, BPW)], pos_v)
    seg_id = base // SEG_LEN
    pltpu.sync_copy(semb_hbm.at[pl.ds(seg_id, 1)], seg_v)
    pltpu.sync_copy(lnw_hbm, lnw_v)
    pltpu.sync_copy(lnb_hbm, lnb_v)

    for j in range(BPW // 16):
        ids16 = idx_v[pl.ds(j * 16, 16)]
        mask_v[pl.ds(j * 16, 16)] = jnp.where(
            ids16 == PAD_WORD, jnp.int32(1), jnp.int32(0))
    pltpu.sync_copy(mask_v, mask_hbm.at[pl.ds(base, BPW)])

    segs = [seg_v[0, pl.ds(c * 16, 16)] for c in range(NCH)]
    lnws = [lnw_v[pl.ds(c * 16, 16)] for c in range(NCH)]
    lnbs = [lnb_v[pl.ds(c * 16, 16)] for c in range(NCH)]
    inv_d = jnp.float32(1.0 / HIDDEN)

    gather.wait()

    lanes = lax.iota(jnp.int32, 16)
    perms = [lax.bitwise_xor(lanes, jnp.int32(s)) for s in (8, 4, 2, 1)]

    def hsum(t):
        for idx in perms:
            t = t + t.at[idx].get(mode="promise_in_bounds",
                                  unique_indices=True)
        return t

    def row_body(i, carry):
        vs = []
        for c in range(NCH):
            w = rows_v[i, pl.ds(c * 16, 16)]
            p = pos_v[i, pl.ds(c * 16, 16)]
            vs.append(w + p + segs[c])
        sq = [v * v for v in vs]
        t0 = (vs[0] + vs[1]) + (vs[2] + vs[3])
        t1 = (vs[4] + vs[5]) + (vs[6] + vs[7])
        s0 = (sq[0] + sq[1]) + (sq[2] + sq[3])
        s1 = (sq[4] + sq[5]) + (sq[6] + sq[7])
        mean = hsum(t0 + t1) * inv_d
        ex2 = hsum(s0 + s1) * inv_d
        x = ex2 - mean * mean + EPS
        iv = lax.bitcast_convert_type(x, jnp.int32)
        y = lax.bitcast_convert_type(
            jnp.int32(MAGIC) - lax.shift_right_logical(iv, 1), jnp.float32)
        half_x = x * 0.5
        for _ in range(3):
            y = y * (1.5 - half_x * y * y)
        neg_my = -(mean * y)
        for c in range(NCH):
            a = y * lnws[c]
            b = neg_my * lnws[c] + lnbs[c]
            rows_v[i, pl.ds(c * 16, 16)] = vs[c] * a + b
        return carry

    lax.fori_loop(0, BPW, row_body, 0, unroll=4)
    pltpu.sync_copy(rows_v, out_hbm.at[pl.ds(base, BPW)])


@jax.jit
def _emb_ln(ids, word_emb, pos_emb, seg_emb, ln_weight, ln_bias):
    mesh = plsc.VectorSubcoreMesh(
        core_axis_name="c", subcore_axis_name="s", num_cores=NC)
    return pl.kernel(
        _emb_ln_body,
        out_type=[
            jax.ShapeDtypeStruct((SEQ, HIDDEN), jnp.float32),
            jax.ShapeDtypeStruct((SEQ,), jnp.int32),
        ],
        mesh=mesh,
        scratch_types=[
            pltpu.VMEM((BPW,), jnp.int32),
            pltpu.VMEM((BPW, HIDDEN), jnp.float32),
            pltpu.VMEM((BPW, HIDDEN), jnp.float32),
            pltpu.VMEM((1, HIDDEN), jnp.float32),
            pltpu.VMEM((HIDDEN,), jnp.float32),
            pltpu.VMEM((HIDDEN,), jnp.float32),
            pltpu.VMEM((BPW,), jnp.int32),
            pltpu.SemaphoreType.DMA,
        ],
    )(ids, word_emb, pos_emb, seg_emb, ln_weight, ln_bias)


def kernel(input_ids_0, input_ids_1, word_emb, pos_emb, seg_emb, ln_weight,
           ln_bias):
    ids = jnp.concatenate([input_ids_0[0], input_ids_1[0, 1:]])
    out, mask = _emb_ln(ids, word_emb, pos_emb, seg_emb, ln_weight, ln_bias)
    return out[None], (mask != 0)[None]
